# Initial kernel scaffold; baseline (speedup 1.0000x reference)
import jax
import jax.numpy as jnp
from jax.experimental import pallas as pl


def kernel(t_in, y_in, rates_1st, rates_2nd, inds_r_1st, inds_p_1st, inds_r1_2nd, inds_r2_2nd, inds_p_2nd, den_norm):
    raise NotImplementedError("write your pallas kernel here")


# speedup vs baseline: 2.8015x; 2.8015x over previous
import functools

import jax
import jax.numpy as jnp
from jax import lax
from jax.experimental import pallas as pl
from jax.experimental.pallas import tpu as pltpu
from jax.experimental.pallas import tpu_sc as plsc

NUM_CORES = 2
NUM_SUBCORES = 16
NUM_WORKERS = NUM_CORES * NUM_SUBCORES
LANES = 16
CHUNK = 4000


def _build_sc_call(B, N, R1, R2):
    rows_per_worker = B // NUM_WORKERS
    span = rows_per_worker * N

    mesh = plsc.VectorSubcoreMesh(
        core_axis_name="c", subcore_axis_name="s",
        num_cores=NUM_CORES, num_subcores=NUM_SUBCORES)

    @functools.partial(
        pl.kernel,
        out_type=jax.ShapeDtypeStruct((B * N,), jnp.float32),
        mesh=mesh,
        compiler_params=pltpu.CompilerParams(needs_layout_passes=False),
        scratch_types=[
            pltpu.VMEM((span,), jnp.float32),
            pltpu.VMEM((span,), jnp.float32),
            pltpu.VMEM((CHUNK,), jnp.int32),
            pltpu.VMEM((CHUNK,), jnp.int32),
            pltpu.VMEM((CHUNK,), jnp.int32),
            pltpu.VMEM((CHUNK,), jnp.float32),
            pltpu.VMEM((LANES,), jnp.float32),
        ],
    )
    def sc_call(y_hbm, k1_hbm, ir1_hbm, ip1_hbm, k2_hbm, ia2_hbm, ib2_hbm,
                ip2_hbm, den_hbm, out_hbm,
                y_v, acc_v, idx_a, idx_b, idx_c, rate_v, den_v):
        wid = lax.axis_index("s") * NUM_CORES + lax.axis_index("c")
        base = wid * span

        pltpu.sync_copy(y_hbm.at[pl.ds(base, span)], y_v)
        pltpu.sync_copy(den_hbm, den_v)
        denv = den_v[...]

        zeros = jnp.zeros((LANES,), jnp.float32)

        def zero_body(i, carry):
            acc_v[pl.ds(i * LANES, LANES)] = zeros
            return carry

        lax.fori_loop(0, span // LANES, zero_body, 0)

        def chunk1(c, carry):
            off = c * CHUNK
            pltpu.sync_copy(k1_hbm.at[pl.ds(off, CHUNK)], rate_v)
            pltpu.sync_copy(ir1_hbm.at[pl.ds(off, CHUNK)], idx_a)
            pltpu.sync_copy(ip1_hbm.at[pl.ds(off, CHUNK)], idx_b)

            def inner(i, icarry):
                s = i * LANES
                irv = idx_a[pl.ds(s, LANES)]
                ipv = idx_b[pl.ds(s, LANES)]
                kv = rate_v[pl.ds(s, LANES)]
                for rr in range(rows_per_worker):
                    o = rr * N
                    t = kv * plsc.load_gather(y_v, [irv + o])
                    plsc.addupdate_scatter(acc_v, [ipv + o], t)
                    plsc.addupdate_scatter(acc_v, [irv + o], -t)
                return icarry

            lax.fori_loop(0, CHUNK // LANES, inner, 0)
            return carry

        lax.fori_loop(0, R1 // CHUNK, chunk1, 0)

        def chunk2(c, carry):
            off = c * CHUNK
            pltpu.sync_copy(k2_hbm.at[pl.ds(off, CHUNK)], rate_v)
            pltpu.sync_copy(ia2_hbm.at[pl.ds(off, CHUNK)], idx_a)
            pltpu.sync_copy(ib2_hbm.at[pl.ds(off, CHUNK)], idx_b)
            pltpu.sync_copy(ip2_hbm.at[pl.ds(off, CHUNK)], idx_c)

            def inner(i, icarry):
                s = i * LANES
                iav = idx_a[pl.ds(s, LANES)]
                ibv = idx_b[pl.ds(s, LANES)]
                ipv = idx_c[pl.ds(s, LANES)]
                kv = rate_v[pl.ds(s, LANES)] * denv
                for rr in range(rows_per_worker):
                    o = rr * N
                    ya = plsc.load_gather(y_v, [iav + o])
                    yb = plsc.load_gather(y_v, [ibv + o])
                    t = kv * ya * yb
                    nt = -t
                    plsc.addupdate_scatter(acc_v, [ipv + o], t)
                    plsc.addupdate_scatter(acc_v, [iav + o], nt)
                    plsc.addupdate_scatter(acc_v, [ibv + o], nt)
                return icarry

            lax.fori_loop(0, CHUNK // LANES, inner, 0)
            return carry

        lax.fori_loop(0, R2 // CHUNK, chunk2, 0)

        pltpu.sync_copy(acc_v, out_hbm.at[pl.ds(base, span)])

    return sc_call


def kernel(t_in, y_in, rates_1st, rates_2nd, inds_r_1st, inds_p_1st,
           inds_r1_2nd, inds_r2_2nd, inds_p_2nd, den_norm):
    del t_in
    B, N = y_in.shape
    R1 = rates_1st.shape[0]
    R2 = rates_2nd.shape[0]
    sc_call = _build_sc_call(B, N, R1, R2)
    den16 = jnp.broadcast_to(den_norm.astype(jnp.float32), (LANES,))
    out_flat = sc_call(
        y_in.reshape(-1),
        rates_1st,
        inds_r_1st.astype(jnp.int32),
        inds_p_1st.astype(jnp.int32),
        rates_2nd,
        inds_r1_2nd.astype(jnp.int32),
        inds_r2_2nd.astype(jnp.int32),
        inds_p_2nd.astype(jnp.int32),
        den16,
    )
    return out_flat.reshape(B, N)


# speedup vs baseline: 3.4685x; 1.2381x over previous
import functools

import jax
import jax.numpy as jnp
from jax import lax
from jax.experimental import pallas as pl
from jax.experimental.pallas import tpu as pltpu
from jax.experimental.pallas import tpu_sc as plsc

NUM_CORES = 2
NUM_SUBCORES = 16
NUM_WORKERS = NUM_CORES * NUM_SUBCORES
LANES = 16
CHUNK = 4000
UNROLL = 5


def _build_sc_call(B, N, R1, R2):
    rows_per_worker = B // NUM_WORKERS
    span = rows_per_worker * N

    mesh = plsc.VectorSubcoreMesh(
        core_axis_name="c", subcore_axis_name="s",
        num_cores=NUM_CORES, num_subcores=NUM_SUBCORES)

    @functools.partial(
        pl.kernel,
        out_type=jax.ShapeDtypeStruct((B * N,), jnp.float32),
        mesh=mesh,
        compiler_params=pltpu.CompilerParams(needs_layout_passes=False),
        scratch_types=[
            pltpu.VMEM((span,), jnp.float32),
            pltpu.VMEM((span,), jnp.float32),
            pltpu.VMEM((2 * CHUNK,), jnp.int32),
            pltpu.VMEM((2 * CHUNK,), jnp.int32),
            pltpu.VMEM((2 * CHUNK,), jnp.int32),
            pltpu.VMEM((2 * CHUNK,), jnp.float32),
            pltpu.VMEM((LANES,), jnp.float32),
            pltpu.SemaphoreType.DMA,
            pltpu.SemaphoreType.DMA,
        ],
    )
    def sc_call(y_hbm, k1_hbm, ir1_hbm, ip1_hbm, k2_hbm, ia2_hbm, ib2_hbm,
                ip2_hbm, den_hbm, out_hbm,
                y_v, acc_v, idx_a, idx_b, idx_c, rate_v, den_v, sem0, sem1):
        wid = lax.axis_index("s") * NUM_CORES + lax.axis_index("c")
        base = wid * span
        sems = [sem0, sem1]

        pltpu.sync_copy(y_hbm.at[pl.ds(base, span)], y_v)
        pltpu.sync_copy(den_hbm, den_v)
        denv = den_v[...]

        zeros = jnp.zeros((LANES,), jnp.float32)

        def zero_body(i, carry):
            acc_v[pl.ds(i * LANES, LANES)] = zeros
            return carry

        lax.fori_loop(0, span // LANES, zero_body, 0)

        def run_phase(n_chunks, hbms, bufs, compute):
            def start(c, slot):
                off = c * CHUNK
                for h, bf in zip(hbms, bufs):
                    pltpu.async_copy(h.at[pl.ds(off, CHUNK)],
                                     bf.at[pl.ds(slot * CHUNK, CHUNK)],
                                     sems[slot])

            def wait(c, slot):
                off = c * CHUNK
                for h, bf in zip(hbms, bufs):
                    pltpu.make_async_copy(h.at[pl.ds(off, CHUNK)],
                                          bf.at[pl.ds(slot * CHUNK, CHUNK)],
                                          sems[slot]).wait()

            start(0, 0)
            start(1, 1)

            def outer(k, carry):
                c0 = k * 2
                for slot in range(2):
                    c = c0 + slot
                    wait(c, slot)
                    compute(slot)

                    @pl.when(c + 2 < n_chunks)
                    def _():
                        start(c + 2, slot)
                return carry

            lax.fori_loop(0, n_chunks // 2, outer, 0)

        def compute1(slot):
            def _b(i, carry):
                s = slot * CHUNK + i * LANES
                irv = idx_a[pl.ds(s, LANES)]
                ipv = idx_b[pl.ds(s, LANES)]
                kv = rate_v[pl.ds(s, LANES)]
                for rr in range(rows_per_worker):
                    o = rr * N
                    t = kv * plsc.load_gather(y_v, [irv + o])
                    plsc.addupdate_scatter(acc_v, [ipv + o], t)
                    plsc.addupdate_scatter(acc_v, [irv + o], -t)
                return carry
            lax.fori_loop(0, CHUNK // LANES, _b, 0)

        run_phase(R1 // CHUNK, [k1_hbm, ir1_hbm, ip1_hbm],
                  [rate_v, idx_a, idx_b], compute1)

        def compute2(slot):
            def _b(i, carry):
                s = slot * CHUNK + i * LANES
                iav = idx_a[pl.ds(s, LANES)]
                ibv = idx_b[pl.ds(s, LANES)]
                ipv = idx_c[pl.ds(s, LANES)]
                kv = rate_v[pl.ds(s, LANES)] * denv
                for rr in range(rows_per_worker):
                    o = rr * N
                    ya = plsc.load_gather(y_v, [iav + o])
                    yb = plsc.load_gather(y_v, [ibv + o])
                    t = kv * ya * yb
                    nt = -t
                    plsc.addupdate_scatter(acc_v, [ipv + o], t)
                    plsc.addupdate_scatter(acc_v, [iav + o], nt)
                    plsc.addupdate_scatter(acc_v, [ibv + o], nt)
                return carry
            lax.fori_loop(0, CHUNK // LANES, _b, 0)

        run_phase(R2 // CHUNK, [k2_hbm, ia2_hbm, ib2_hbm, ip2_hbm],
                  [rate_v, idx_a, idx_b, idx_c], compute2)

        pltpu.sync_copy(acc_v, out_hbm.at[pl.ds(base, span)])

    return sc_call


def kernel(t_in, y_in, rates_1st, rates_2nd, inds_r_1st, inds_p_1st,
           inds_r1_2nd, inds_r2_2nd, inds_p_2nd, den_norm):
    del t_in
    B, N = y_in.shape
    R1 = rates_1st.shape[0]
    R2 = rates_2nd.shape[0]
    sc_call = _build_sc_call(B, N, R1, R2)
    den16 = jnp.broadcast_to(den_norm.astype(jnp.float32), (LANES,))
    out_flat = sc_call(
        y_in.reshape(-1),
        rates_1st,
        inds_r_1st.astype(jnp.int32),
        inds_p_1st.astype(jnp.int32),
        rates_2nd,
        inds_r1_2nd.astype(jnp.int32),
        inds_r2_2nd.astype(jnp.int32),
        inds_p_2nd.astype(jnp.int32),
        den16,
    )
    return out_flat.reshape(B, N)


# speedup vs baseline: 6.5018x; 1.8745x over previous
import functools

import jax
import jax.numpy as jnp
from jax import lax
from jax.experimental import pallas as pl
from jax.experimental.pallas import tpu as pltpu
from jax.experimental.pallas import tpu_sc as plsc

NUM_CORES = 2
NUM_SUBCORES = 16
NUM_WORKERS = NUM_CORES * NUM_SUBCORES
LANES = 16
CHUNK = 3200
G = 4


def _build_sc_call(B, N, R1, R2):
    rows_per_worker = B // NUM_WORKERS
    span = rows_per_worker * N

    mesh = plsc.VectorSubcoreMesh(
        core_axis_name="c", subcore_axis_name="s",
        num_cores=NUM_CORES, num_subcores=NUM_SUBCORES)

    @functools.partial(
        pl.kernel,
        out_type=jax.ShapeDtypeStruct((B * N,), jnp.float32),
        mesh=mesh,
        compiler_params=pltpu.CompilerParams(needs_layout_passes=False),
        scratch_types=[
            pltpu.VMEM((span,), jnp.float32),
            pltpu.VMEM((span,), jnp.float32),
            pltpu.VMEM((2 * CHUNK,), jnp.int32),
            pltpu.VMEM((2 * CHUNK,), jnp.int32),
            pltpu.VMEM((2 * CHUNK,), jnp.int32),
            pltpu.VMEM((2 * CHUNK,), jnp.float32),
            pltpu.VMEM((LANES,), jnp.float32),
            pltpu.SemaphoreType.DMA,
            pltpu.SemaphoreType.DMA,
        ],
    )
    def sc_call(y_hbm, k1_hbm, ir1_hbm, ip1_hbm, k2_hbm, ia2_hbm, ib2_hbm,
                ip2_hbm, den_hbm, out_hbm,
                y_v, acc_v, idx_a, idx_b, idx_c, rate_v, den_v, sem0, sem1):
        wid = lax.axis_index("s") * NUM_CORES + lax.axis_index("c")
        base = wid * span
        sems = [sem0, sem1]

        pltpu.sync_copy(y_hbm.at[pl.ds(base, span)], y_v)
        pltpu.sync_copy(den_hbm, den_v)
        denv = den_v[...]

        zeros = jnp.zeros((LANES,), jnp.float32)

        def zero_body(i, carry):
            acc_v[pl.ds(i * LANES, LANES)] = zeros
            return carry

        lax.fori_loop(0, span // LANES, zero_body, 0)

        def run_phase(n_chunks, hbms, bufs, compute):
            def start(c, slot):
                off = c * CHUNK
                for h, bf in zip(hbms, bufs):
                    pltpu.async_copy(h.at[pl.ds(off, CHUNK)],
                                     bf.at[pl.ds(slot * CHUNK, CHUNK)],
                                     sems[slot])

            def wait(c, slot):
                off = c * CHUNK
                for h, bf in zip(hbms, bufs):
                    pltpu.make_async_copy(h.at[pl.ds(off, CHUNK)],
                                          bf.at[pl.ds(slot * CHUNK, CHUNK)],
                                          sems[slot]).wait()

            start(0, 0)
            start(1, 1)

            def outer(k, carry):
                c0 = k * 2
                for slot in range(2):
                    c = c0 + slot
                    wait(c, slot)
                    compute(slot)

                    @pl.when(c + 2 < n_chunks)
                    def _():
                        start(c + 2, slot)
                return carry

            lax.fori_loop(0, n_chunks // 2, outer, 0)

        def compute1(slot):
            def _b(i, carry):
                s = slot * CHUNK + i * (LANES * G)
                vecs = []
                for g in range(G):
                    sg = s + g * LANES
                    irv = idx_a[pl.ds(sg, LANES)]
                    ipv = idx_b[pl.ds(sg, LANES)]
                    kv = rate_v[pl.ds(sg, LANES)]
                    vecs.append((irv, ipv, kv))
                for rr in range(rows_per_worker):
                    o = rr * N
                    ts = [kv * plsc.load_gather(y_v, [irv + o])
                          for (irv, ipv, kv) in vecs]
                    for t, (irv, ipv, kv) in zip(ts, vecs):
                        plsc.addupdate_scatter(acc_v, [ipv + o], t)
                        plsc.addupdate_scatter(acc_v, [irv + o], -t)
                return carry

            lax.fori_loop(0, CHUNK // (LANES * G), _b, 0)

        run_phase(R1 // CHUNK, [k1_hbm, ir1_hbm, ip1_hbm],
                  [rate_v, idx_a, idx_b], compute1)

        def compute2(slot):
            def _b(i, carry):
                s = slot * CHUNK + i * (LANES * G)
                vecs = []
                for g in range(G):
                    sg = s + g * LANES
                    iav = idx_a[pl.ds(sg, LANES)]
                    ibv = idx_b[pl.ds(sg, LANES)]
                    ipv = idx_c[pl.ds(sg, LANES)]
                    kv = rate_v[pl.ds(sg, LANES)] * denv
                    vecs.append((iav, ibv, ipv, kv))
                for rr in range(rows_per_worker):
                    o = rr * N
                    ts = [kv * plsc.load_gather(y_v, [iav + o])
                          * plsc.load_gather(y_v, [ibv + o])
                          for (iav, ibv, ipv, kv) in vecs]
                    for t, (iav, ibv, ipv, kv) in zip(ts, vecs):
                        nt = -t
                        plsc.addupdate_scatter(acc_v, [ipv + o], t)
                        plsc.addupdate_scatter(acc_v, [iav + o], nt)
                        plsc.addupdate_scatter(acc_v, [ibv + o], nt)
                return carry

            lax.fori_loop(0, CHUNK // (LANES * G), _b, 0)

        run_phase(R2 // CHUNK, [k2_hbm, ia2_hbm, ib2_hbm, ip2_hbm],
                  [rate_v, idx_a, idx_b, idx_c], compute2)

        pltpu.sync_copy(acc_v, out_hbm.at[pl.ds(base, span)])

    return sc_call


def kernel(t_in, y_in, rates_1st, rates_2nd, inds_r_1st, inds_p_1st,
           inds_r1_2nd, inds_r2_2nd, inds_p_2nd, den_norm):
    del t_in
    B, N = y_in.shape
    R1 = rates_1st.shape[0]
    R2 = rates_2nd.shape[0]
    sc_call = _build_sc_call(B, N, R1, R2)
    den16 = jnp.broadcast_to(den_norm.astype(jnp.float32), (LANES,))
    out_flat = sc_call(
        y_in.reshape(-1),
        rates_1st,
        inds_r_1st.astype(jnp.int32),
        inds_p_1st.astype(jnp.int32),
        rates_2nd,
        inds_r1_2nd.astype(jnp.int32),
        inds_r2_2nd.astype(jnp.int32),
        inds_p_2nd.astype(jnp.int32),
        den16,
    )
    return out_flat.reshape(B, N)


# speedup vs baseline: 6.6477x; 1.0224x over previous
import functools

import jax
import jax.numpy as jnp
from jax import lax
from jax.experimental import pallas as pl
from jax.experimental.pallas import tpu as pltpu
from jax.experimental.pallas import tpu_sc as plsc

NUM_CORES = 2
NUM_SUBCORES = 16
NUM_WORKERS = NUM_CORES * NUM_SUBCORES
LANES = 16
CHUNK = 3200
G = 8


def _build_sc_call(B, N, R1, R2):
    rows_per_worker = B // NUM_WORKERS
    span = rows_per_worker * N

    mesh = plsc.VectorSubcoreMesh(
        core_axis_name="c", subcore_axis_name="s",
        num_cores=NUM_CORES, num_subcores=NUM_SUBCORES)

    @functools.partial(
        pl.kernel,
        out_type=jax.ShapeDtypeStruct((B * N,), jnp.float32),
        mesh=mesh,
        compiler_params=pltpu.CompilerParams(needs_layout_passes=False),
        scratch_types=[
            pltpu.VMEM((span,), jnp.float32),
            pltpu.VMEM((span,), jnp.float32),
            pltpu.VMEM((2 * CHUNK,), jnp.int32),
            pltpu.VMEM((2 * CHUNK,), jnp.int32),
            pltpu.VMEM((2 * CHUNK,), jnp.int32),
            pltpu.VMEM((2 * CHUNK,), jnp.float32),
            pltpu.VMEM((LANES,), jnp.float32),
            pltpu.SemaphoreType.DMA,
            pltpu.SemaphoreType.DMA,
        ],
    )
    def sc_call(y_hbm, k1_hbm, ir1_hbm, ip1_hbm, k2_hbm, ia2_hbm, ib2_hbm,
                ip2_hbm, den_hbm, out_hbm,
                y_v, acc_v, idx_a, idx_b, idx_c, rate_v, den_v, sem0, sem1):
        wid = lax.axis_index("s") * NUM_CORES + lax.axis_index("c")
        base = wid * span
        sems = [sem0, sem1]

        pltpu.sync_copy(y_hbm.at[pl.ds(base, span)], y_v)
        pltpu.sync_copy(den_hbm, den_v)
        denv = den_v[...]

        zeros = jnp.zeros((LANES,), jnp.float32)

        def zero_body(i, carry):
            acc_v[pl.ds(i * LANES, LANES)] = zeros
            return carry

        lax.fori_loop(0, span // LANES, zero_body, 0)

        def run_phase(n_chunks, hbms, bufs, compute):
            def start(c, slot):
                off = c * CHUNK
                for h, bf in zip(hbms, bufs):
                    pltpu.async_copy(h.at[pl.ds(off, CHUNK)],
                                     bf.at[pl.ds(slot * CHUNK, CHUNK)],
                                     sems[slot])

            def wait(c, slot):
                off = c * CHUNK
                for h, bf in zip(hbms, bufs):
                    pltpu.make_async_copy(h.at[pl.ds(off, CHUNK)],
                                          bf.at[pl.ds(slot * CHUNK, CHUNK)],
                                          sems[slot]).wait()

            start(0, 0)
            start(1, 1)

            def outer(k, carry):
                c0 = k * 2
                for slot in range(2):
                    c = c0 + slot
                    wait(c, slot)
                    compute(slot)

                    @pl.when(c + 2 < n_chunks)
                    def _():
                        start(c + 2, slot)
                return carry

            lax.fori_loop(0, n_chunks // 2, outer, 0)

        def compute1(slot):
            def _b(i, carry):
                s = slot * CHUNK + i * (LANES * G)
                vecs = []
                for g in range(G):
                    sg = s + g * LANES
                    irv = idx_a[pl.ds(sg, LANES)]
                    ipv = idx_b[pl.ds(sg, LANES)]
                    kv = rate_v[pl.ds(sg, LANES)]
                    vecs.append((irv, ipv, kv))
                for rr in range(rows_per_worker):
                    o = rr * N
                    ts = [kv * plsc.load_gather(y_v, [irv + o])
                          for (irv, ipv, kv) in vecs]
                    for t, (irv, ipv, kv) in zip(ts, vecs):
                        plsc.addupdate_scatter(acc_v, [ipv + o], t)
                        plsc.addupdate_scatter(acc_v, [irv + o], -t)
                return carry

            lax.fori_loop(0, CHUNK // (LANES * G), _b, 0)

        run_phase(R1 // CHUNK, [k1_hbm, ir1_hbm, ip1_hbm],
                  [rate_v, idx_a, idx_b], compute1)

        def compute2(slot):
            def _b(i, carry):
                s = slot * CHUNK + i * (LANES * G)
                vecs = []
                for g in range(G):
                    sg = s + g * LANES
                    iav = idx_a[pl.ds(sg, LANES)]
                    ibv = idx_b[pl.ds(sg, LANES)]
                    ipv = idx_c[pl.ds(sg, LANES)]
                    kv = rate_v[pl.ds(sg, LANES)] * denv
                    vecs.append((iav, ibv, ipv, kv))
                for rr in range(rows_per_worker):
                    o = rr * N
                    ts = [kv * plsc.load_gather(y_v, [iav + o])
                          * plsc.load_gather(y_v, [ibv + o])
                          for (iav, ibv, ipv, kv) in vecs]
                    for t, (iav, ibv, ipv, kv) in zip(ts, vecs):
                        nt = -t
                        plsc.addupdate_scatter(acc_v, [ipv + o], t)
                        plsc.addupdate_scatter(acc_v, [iav + o], nt)
                        plsc.addupdate_scatter(acc_v, [ibv + o], nt)
                return carry

            lax.fori_loop(0, CHUNK // (LANES * G), _b, 0)

        run_phase(R2 // CHUNK, [k2_hbm, ia2_hbm, ib2_hbm, ip2_hbm],
                  [rate_v, idx_a, idx_b, idx_c], compute2)

        pltpu.sync_copy(acc_v, out_hbm.at[pl.ds(base, span)])

    return sc_call


def kernel(t_in, y_in, rates_1st, rates_2nd, inds_r_1st, inds_p_1st,
           inds_r1_2nd, inds_r2_2nd, inds_p_2nd, den_norm):
    del t_in
    B, N = y_in.shape
    R1 = rates_1st.shape[0]
    R2 = rates_2nd.shape[0]
    sc_call = _build_sc_call(B, N, R1, R2)
    den16 = jnp.broadcast_to(den_norm.astype(jnp.float32), (LANES,))
    out_flat = sc_call(
        y_in.reshape(-1),
        rates_1st,
        inds_r_1st.astype(jnp.int32),
        inds_p_1st.astype(jnp.int32),
        rates_2nd,
        inds_r1_2nd.astype(jnp.int32),
        inds_r2_2nd.astype(jnp.int32),
        inds_p_2nd.astype(jnp.int32),
        den16,
    )
    return out_flat.reshape(B, N)


# speedup vs baseline: 7.1843x; 1.0807x over previous
import functools

import jax
import jax.numpy as jnp
from jax import lax
from jax.experimental import pallas as pl
from jax.experimental.pallas import tpu as pltpu
from jax.experimental.pallas import tpu_sc as plsc

NUM_CORES = 2
NUM_SUBCORES = 16
NUM_WORKERS = NUM_CORES * NUM_SUBCORES
LANES = 16
CHUNK = 3200
G = 8


def _build_sc_call(B, N, R1, R2):
    rows_per_worker = B // NUM_WORKERS
    span = rows_per_worker * N

    mesh = plsc.VectorSubcoreMesh(
        core_axis_name="c", subcore_axis_name="s",
        num_cores=NUM_CORES, num_subcores=NUM_SUBCORES)

    @functools.partial(
        pl.kernel,
        out_type=jax.ShapeDtypeStruct((B * N,), jnp.float32),
        mesh=mesh,
        compiler_params=pltpu.CompilerParams(needs_layout_passes=False),
        scratch_types=[
            pltpu.VMEM((span,), jnp.float32),
            pltpu.VMEM((span,), jnp.float32),
            pltpu.VMEM((2 * CHUNK,), jnp.int32),
            pltpu.VMEM((2 * CHUNK,), jnp.int32),
            pltpu.VMEM((2 * CHUNK,), jnp.int32),
            pltpu.VMEM((2 * CHUNK,), jnp.float32),
            pltpu.VMEM((LANES,), jnp.float32),
            pltpu.VMEM((N,), jnp.float32),
            pltpu.SemaphoreType.DMA,
            pltpu.SemaphoreType.DMA,
        ],
    )
    def sc_call(y_hbm, k1_hbm, ir1_hbm, ip1_hbm, k2_hbm, ia2_hbm, ib2_hbm,
                ip2_hbm, den_hbm, out_hbm,
                y_v, acc_v, idx_a, idx_b, idx_c, rate_v, den_v, c1_v,
                sem0, sem1):
        wid = lax.axis_index("s") * NUM_CORES + lax.axis_index("c")
        base = wid * span
        sems = [sem0, sem1]

        pltpu.sync_copy(y_hbm.at[pl.ds(base, span)], y_v)
        pltpu.sync_copy(den_hbm, den_v)
        denv = den_v[...]

        zeros = jnp.zeros((LANES,), jnp.float32)

        def zero_body(i, carry):
            acc_v[pl.ds(i * LANES, LANES)] = zeros
            return carry

        lax.fori_loop(0, span // LANES, zero_body, 0)

        def zero_c1(i, carry):
            c1_v[pl.ds(i * LANES, LANES)] = zeros
            return carry

        lax.fori_loop(0, N // LANES, zero_c1, 0)

        def run_phase(n_chunks, hbms, bufs, compute):
            def start(c, slot):
                off = c * CHUNK
                for h, bf in zip(hbms, bufs):
                    pltpu.async_copy(h.at[pl.ds(off, CHUNK)],
                                     bf.at[pl.ds(slot * CHUNK, CHUNK)],
                                     sems[slot])

            def wait(c, slot):
                off = c * CHUNK
                for h, bf in zip(hbms, bufs):
                    pltpu.make_async_copy(h.at[pl.ds(off, CHUNK)],
                                          bf.at[pl.ds(slot * CHUNK, CHUNK)],
                                          sems[slot]).wait()

            start(0, 0)
            start(1, 1)

            def outer(k, carry):
                c0 = k * 2
                for slot in range(2):
                    c = c0 + slot
                    wait(c, slot)
                    compute(slot)

                    @pl.when(c + 2 < n_chunks)
                    def _():
                        start(c + 2, slot)
                return carry

            lax.fori_loop(0, n_chunks // 2, outer, 0)

        def compute1(slot):
            def _b(i, carry):
                s = slot * CHUNK + i * (LANES * G)
                vecs = []
                for g in range(G):
                    sg = s + g * LANES
                    irv = idx_a[pl.ds(sg, LANES)]
                    ipv = idx_b[pl.ds(sg, LANES)]
                    kv = rate_v[pl.ds(sg, LANES)]
                    vecs.append((irv, ipv, kv))
                for (irv, ipv, kv) in vecs:
                    plsc.addupdate_scatter(c1_v, [irv], kv)
                for rr in range(rows_per_worker):
                    yr = y_v.at[pl.ds(rr * N, N)]
                    ar = acc_v.at[pl.ds(rr * N, N)]
                    ts = [kv * plsc.load_gather(yr, [irv])
                          for (irv, ipv, kv) in vecs]
                    for t, (irv, ipv, kv) in zip(ts, vecs):
                        plsc.addupdate_scatter(ar, [ipv], t)
                return carry

            lax.fori_loop(0, CHUNK // (LANES * G), _b, 0)

        run_phase(R1 // CHUNK, [k1_hbm, ir1_hbm, ip1_hbm],
                  [rate_v, idx_a, idx_b], compute1)

        def compute2(slot):
            def _b(i, carry):
                s = slot * CHUNK + i * (LANES * G)
                vecs = []
                for g in range(G):
                    sg = s + g * LANES
                    iav = idx_a[pl.ds(sg, LANES)]
                    ibv = idx_b[pl.ds(sg, LANES)]
                    ipv = idx_c[pl.ds(sg, LANES)]
                    kv = rate_v[pl.ds(sg, LANES)] * denv
                    vecs.append((iav, ibv, ipv, kv))
                for rr in range(rows_per_worker):
                    yr = y_v.at[pl.ds(rr * N, N)]
                    ar = acc_v.at[pl.ds(rr * N, N)]
                    ts = [kv * plsc.load_gather(yr, [iav])
                          * plsc.load_gather(yr, [ibv])
                          for (iav, ibv, ipv, kv) in vecs]
                    for t, (iav, ibv, ipv, kv) in zip(ts, vecs):
                        nt = -t
                        plsc.addupdate_scatter(ar, [ipv], t)
                        plsc.addupdate_scatter(ar, [iav], nt)
                        plsc.addupdate_scatter(ar, [ibv], nt)
                return carry

            lax.fori_loop(0, CHUNK // (LANES * G), _b, 0)

        run_phase(R2 // CHUNK, [k2_hbm, ia2_hbm, ib2_hbm, ip2_hbm],
                  [rate_v, idx_a, idx_b, idx_c], compute2)

        def dense_destruct(j, carry):
            sj = j * LANES
            cv = c1_v[pl.ds(sj, LANES)]
            for rr in range(rows_per_worker):
                o = rr * N + sj
                acc_v[pl.ds(o, LANES)] = (acc_v[pl.ds(o, LANES)]
                                          - y_v[pl.ds(o, LANES)] * cv)
            return carry

        lax.fori_loop(0, N // LANES, dense_destruct, 0)

        pltpu.sync_copy(acc_v, out_hbm.at[pl.ds(base, span)])

    return sc_call


def kernel(t_in, y_in, rates_1st, rates_2nd, inds_r_1st, inds_p_1st,
           inds_r1_2nd, inds_r2_2nd, inds_p_2nd, den_norm):
    del t_in
    B, N = y_in.shape
    R1 = rates_1st.shape[0]
    R2 = rates_2nd.shape[0]
    sc_call = _build_sc_call(B, N, R1, R2)
    den16 = jnp.broadcast_to(den_norm.astype(jnp.float32), (LANES,))
    out_flat = sc_call(
        y_in.reshape(-1),
        rates_1st,
        inds_r_1st.astype(jnp.int32),
        inds_p_1st.astype(jnp.int32),
        rates_2nd,
        inds_r1_2nd.astype(jnp.int32),
        inds_r2_2nd.astype(jnp.int32),
        inds_p_2nd.astype(jnp.int32),
        den16,
    )
    return out_flat.reshape(B, N)
